# baseline (speedup 1.0000x reference)
import functools

import jax
import jax.numpy as jnp
from jax import lax
from jax.experimental import pallas as pl
from jax.experimental.pallas import tpu as pltpu
from jax.experimental.pallas import tpu_sc as plsc

B, H, W, C = 4, 112, 112, 192
oH, oW = 2 * H, 2 * W
CB = 16
NCB = C // CB
NC, NS = 2, 16
NTASK = B * NCB
TPC = NTASK // NC
HPT = H // NS
EPT = HPT * CB * W
NCHUNK = EPT // 128
ACC = oH * CB * oW
OPT = ACC // NS
YPT = oH // NS
WCH = CB * oW
NWP = YPT // 2
ZCH = 3136

_mesh = plsc.VectorSubcoreMesh(core_axis_name="c", subcore_axis_name="s")


@functools.partial(
    pl.kernel,
    mesh=_mesh,
    out_type=jax.ShapeDtypeStruct((B, oH, C, oW), jnp.float32),
    scratch_types=[
        pltpu.VMEM((HPT, CB, W), jnp.float32),
        pltpu.VMEM((HPT, CB, W), jnp.int32),
        pltpu.VMEM((NCHUNK, 128), jnp.float32),
        pltpu.VMEM((NCHUNK, 128), jnp.int32),
        pltpu.VMEM((WCH,), jnp.float32),
        pltpu.VMEM((WCH,), jnp.float32),
        pltpu.VMEM((1, CB, oW), jnp.float32),
        pltpu.VMEM((1, CB, oW), jnp.float32),
        pltpu.VMEM((ZCH,), jnp.float32),
        pltpu.VMEM_SHARED((ACC,), jnp.float32),
        pltpu.SemaphoreType.DMA,
        pltpu.SemaphoreType.DMA,
        pltpu.SemaphoreType.DMA,
        pltpu.SemaphoreType.DMA,
        pltpu.SemaphoreType.DMA,
        pltpu.SemaphoreType.DMA,
        pltpu.SemaphoreType.DMA,
    ],
)
def _unpool_sc(upd_hbm, mask_hbm, out_hbm, u_raw, m_raw, uv, iv, st1a, st1b,
               st3a, st3b, zbuf, acc, sem_s, sem_z, sem_in, sem_ra, sem_rb,
               sem_wa, sem_wb):
    core = lax.axis_index("c")
    sid = lax.axis_index("s")

    zero16 = jnp.zeros((16,), jnp.float32)

    def zinit(i, carry):
        zbuf[pl.ds(i * 16, 16)] = zero16
        return carry

    lax.fori_loop(0, ZCH // 16, zinit, 0)

    third = jnp.float32(1.0 / 3.0)
    seventh = jnp.float32(1.0 / 7.0)

    h0 = sid * HPT
    y0 = sid * YPT

    def in_slices(task):
        b = task // NCB
        cb0 = (task % NCB) * CB
        return (upd_hbm.at[b, pl.ds(h0, HPT), pl.ds(cb0, CB), :],
                mask_hbm.at[b, pl.ds(h0, HPT), pl.ds(cb0, CB), :])

    u_sl0, m_sl0 = in_slices(core * TPC)
    pltpu.async_copy(u_sl0, u_raw, sem_in)
    pltpu.async_copy(m_sl0, m_raw, sem_in)

    def task_body(t, carry):
        task = core * TPC + t
        b = task // NCB
        cb0 = (task % NCB) * CB

        for z in range(OPT // ZCH):
            pltpu.async_copy(zbuf, acc.at[pl.ds(sid * OPT + z * ZCH, ZCH)],
                             sem_z)

        u_sl, m_sl = in_slices(task)
        pltpu.make_async_copy(u_sl, u_raw, sem_in).wait()
        pltpu.make_async_copy(m_sl, m_raw, sem_in).wait()

        @plsc.parallel_loop(0, HPT * CB, unroll=4)
        def _compute(row):
            hh = row >> 4
            cc = row & (CB - 1)
            c224 = cc * oW
            j0 = row * (W // 16)
            for v in range(W // 16):
                m = m_raw[hh, cc, pl.ds(v * 16, 16)]
                u = u_raw[hh, cc, pl.ds(v * 16, 16)]
                t6 = lax.shift_right_logical(m, 6)
                s = (t6.astype(jnp.float32) * third).astype(jnp.int32)
                t7 = lax.shift_right_logical(s, 5)
                y = (t7.astype(jnp.float32) * seventh).astype(jnp.int32)
                idx = s + y * (CB * oW - oW) + c224
                j = j0 + v
                cj = j >> 3
                off = (j & 7) * 16
                iv[cj, pl.ds(off, 16)] = idx
                uv[cj, pl.ds(off, 16)] = u

        for z in range(OPT // ZCH):
            pltpu.make_async_copy(
                zbuf, acc.at[pl.ds(sid * OPT + z * ZCH, ZCH)], sem_z).wait()
        plsc.subcore_barrier()

        def cscat(cj, carry2):
            pltpu.async_copy(uv.at[cj], acc.at[iv.at[cj]], sem_s, add=True)
            return carry2

        lax.fori_loop(0, NCHUNK, cscat, 0)

        @pl.when(t < TPC - 1)
        def _prefetch():
            u_sn, m_sn = in_slices(task + 1)
            pltpu.async_copy(u_sn, u_raw, sem_in)
            pltpu.async_copy(m_sn, m_raw, sem_in)

        def cdrain(cj, carry2):
            pltpu.make_async_copy(uv.at[cj], acc.at[iv.at[cj]], sem_s).wait()
            return carry2

        lax.fori_loop(0, NCHUNK, cdrain, 0)
        plsc.subcore_barrier()

        def acc_sl(r):
            return acc.at[pl.ds(sid * OPT + r * WCH, WCH)]

        def out_sl(r):
            return out_hbm.at[b, pl.ds(y0 + r, 1), pl.ds(cb0, CB), :]

        def regroup(st1, st3):
            @plsc.parallel_loop(0, CB, unroll=2)
            def _wc(cc):
                base = cc * oW
                for v in range(oW // 16):
                    st3[0, cc, pl.ds(v * 16, 16)] = (
                        st1[pl.ds(base + v * 16, 16)])

        pltpu.async_copy(acc_sl(0), st1a, sem_ra)

        def wpair(p, carry2):
            r0 = 2 * p
            pltpu.make_async_copy(acc_sl(r0), st1a, sem_ra).wait()
            pltpu.async_copy(acc_sl(r0 + 1), st1b, sem_rb)

            @pl.when(p > 0)
            def _wa():
                pltpu.make_async_copy(st3a, out_sl(r0 - 2), sem_wa).wait()

            regroup(st1a, st3a)
            pltpu.async_copy(st3a, out_sl(r0), sem_wa)

            @pl.when(p < NWP - 1)
            def _ra():
                pltpu.async_copy(acc_sl(r0 + 2), st1a, sem_ra)

            pltpu.make_async_copy(acc_sl(r0 + 1), st1b, sem_rb).wait()

            @pl.when(p > 0)
            def _wb():
                pltpu.make_async_copy(st3b, out_sl(r0 - 1), sem_wb).wait()

            regroup(st1b, st3b)
            pltpu.async_copy(st3b, out_sl(r0 + 1), sem_wb)
            return carry2

        lax.fori_loop(0, NWP, wpair, 0)
        pltpu.make_async_copy(st3a, out_sl(YPT - 2), sem_wa).wait()
        pltpu.make_async_copy(st3b, out_sl(YPT - 1), sem_wb).wait()
        return carry

    lax.fori_loop(0, TPC, task_body, 0)


def kernel(updates, mask):
    u4 = updates.transpose(0, 1, 3, 2)
    m4 = mask.astype(jnp.int32).transpose(0, 1, 3, 2)
    out = _unpool_sc(u4, m4)
    return out.transpose(0, 1, 3, 2)


# speedup vs baseline: 1.0003x; 1.0003x over previous
import functools

import jax
import jax.numpy as jnp
from jax import lax
from jax.experimental import pallas as pl
from jax.experimental.pallas import tpu as pltpu
from jax.experimental.pallas import tpu_sc as plsc

B, H, W, C = 4, 112, 112, 192
oH, oW = 2 * H, 2 * W
CB = 16
NCB = C // CB
NC, NS = 2, 16
NTASK = B * NCB
TPC = NTASK // NC
HPT = H // NS
EPT = HPT * CB * W
NCHUNK = EPT // 128
ACC = oH * CB * oW
OPT = ACC // NS
YPT = oH // NS
WCH = CB * oW
NWP = YPT // 2
ZCH = 3136

_mesh = plsc.VectorSubcoreMesh(core_axis_name="c", subcore_axis_name="s")


@functools.partial(
    pl.kernel,
    mesh=_mesh,
    out_type=jax.ShapeDtypeStruct((B, oH, C, oW), jnp.float32),
    scratch_types=[
        pltpu.VMEM((HPT, CB, W), jnp.float32),
        pltpu.VMEM((HPT, CB, W), jnp.int32),
        pltpu.VMEM((NCHUNK, 128), jnp.float32),
        pltpu.VMEM((NCHUNK, 128), jnp.int32),
        pltpu.VMEM((WCH,), jnp.float32),
        pltpu.VMEM((WCH,), jnp.float32),
        pltpu.VMEM((1, CB, oW), jnp.float32),
        pltpu.VMEM((1, CB, oW), jnp.float32),
        pltpu.VMEM((ZCH,), jnp.float32),
        pltpu.VMEM_SHARED((ACC,), jnp.float32),
        pltpu.SemaphoreType.DMA,
        pltpu.SemaphoreType.DMA,
        pltpu.SemaphoreType.DMA,
        pltpu.SemaphoreType.DMA,
        pltpu.SemaphoreType.DMA,
        pltpu.SemaphoreType.DMA,
        pltpu.SemaphoreType.DMA,
    ],
)
def _unpool_sc(upd_hbm, mask_hbm, out_hbm, u_raw, m_raw, uv, iv, st1a, st1b,
               st3a, st3b, zbuf, acc, sem_s, sem_z, sem_in, sem_ra, sem_rb,
               sem_wa, sem_wb):
    core = lax.axis_index("c")
    sid = lax.axis_index("s")

    zero16 = jnp.zeros((16,), jnp.float32)

    def zinit(i, carry):
        zbuf[pl.ds(i * 16, 16)] = zero16
        return carry

    lax.fori_loop(0, ZCH // 16, zinit, 0)

    third = jnp.float32(1.0 / 3.0)
    seventh = jnp.float32(1.0 / 7.0)

    h0 = sid * HPT
    y0 = sid * YPT

    def in_slices(task):
        b = task // NCB
        cb0 = (task % NCB) * CB
        return (upd_hbm.at[b, pl.ds(h0, HPT), pl.ds(cb0, CB), :],
                mask_hbm.at[b, pl.ds(h0, HPT), pl.ds(cb0, CB), :])

    u_sl0, m_sl0 = in_slices(core * TPC)
    pltpu.async_copy(u_sl0, u_raw, sem_in)
    pltpu.async_copy(m_sl0, m_raw, sem_in)

    def task_body(t, carry):
        task = core * TPC + t
        b = task // NCB
        cb0 = (task % NCB) * CB

        for z in range(OPT // ZCH):
            pltpu.async_copy(zbuf, acc.at[pl.ds(sid * OPT + z * ZCH, ZCH)],
                             sem_z)

        u_sl, m_sl = in_slices(task)
        pltpu.make_async_copy(u_sl, u_raw, sem_in).wait()
        pltpu.make_async_copy(m_sl, m_raw, sem_in).wait()

        @plsc.parallel_loop(0, HPT * CB, unroll=4)
        def _compute(row):
            hh = row >> 4
            cc = row & (CB - 1)
            c224 = cc * oW
            j0 = row * (W // 16)
            for v in range(W // 16):
                m = m_raw[hh, cc, pl.ds(v * 16, 16)]
                u = u_raw[hh, cc, pl.ds(v * 16, 16)]
                t6 = lax.shift_right_logical(m, 6)
                s = (t6.astype(jnp.float32) * third).astype(jnp.int32)
                t7 = lax.shift_right_logical(s, 5)
                y = (t7.astype(jnp.float32) * seventh).astype(jnp.int32)
                idx = s + y * (CB * oW - oW) + c224
                j = j0 + v
                cj = j >> 3
                off = (j & 7) * 16
                iv[cj, pl.ds(off, 16)] = idx
                uv[cj, pl.ds(off, 16)] = u

        for z in range(OPT // ZCH):
            pltpu.make_async_copy(
                zbuf, acc.at[pl.ds(sid * OPT + z * ZCH, ZCH)], sem_z).wait()
        plsc.subcore_barrier()

        @plsc.parallel_loop(0, NCHUNK, unroll=4)
        def _cscat(cj):
            pltpu.async_copy(uv.at[cj], acc.at[iv.at[cj]], sem_s, add=True)

        @pl.when(t < TPC - 1)
        def _prefetch():
            u_sn, m_sn = in_slices(task + 1)
            pltpu.async_copy(u_sn, u_raw, sem_in)
            pltpu.async_copy(m_sn, m_raw, sem_in)

        @plsc.parallel_loop(0, NCHUNK, unroll=4)
        def _cdrain(cj):
            pltpu.make_async_copy(uv.at[cj], acc.at[iv.at[cj]], sem_s).wait()
        plsc.subcore_barrier()

        def acc_sl(r):
            return acc.at[pl.ds(sid * OPT + r * WCH, WCH)]

        def out_sl(r):
            return out_hbm.at[b, pl.ds(y0 + r, 1), pl.ds(cb0, CB), :]

        def regroup(st1, st3):
            @plsc.parallel_loop(0, CB, unroll=2)
            def _wc(cc):
                base = cc * oW
                for v in range(oW // 16):
                    st3[0, cc, pl.ds(v * 16, 16)] = (
                        st1[pl.ds(base + v * 16, 16)])

        pltpu.async_copy(acc_sl(0), st1a, sem_ra)

        def wpair(p, carry2):
            r0 = 2 * p
            pltpu.make_async_copy(acc_sl(r0), st1a, sem_ra).wait()
            pltpu.async_copy(acc_sl(r0 + 1), st1b, sem_rb)

            @pl.when(p > 0)
            def _wa():
                pltpu.make_async_copy(st3a, out_sl(r0 - 2), sem_wa).wait()

            regroup(st1a, st3a)
            pltpu.async_copy(st3a, out_sl(r0), sem_wa)

            @pl.when(p < NWP - 1)
            def _ra():
                pltpu.async_copy(acc_sl(r0 + 2), st1a, sem_ra)

            pltpu.make_async_copy(acc_sl(r0 + 1), st1b, sem_rb).wait()

            @pl.when(p > 0)
            def _wb():
                pltpu.make_async_copy(st3b, out_sl(r0 - 1), sem_wb).wait()

            regroup(st1b, st3b)
            pltpu.async_copy(st3b, out_sl(r0 + 1), sem_wb)
            return carry2

        lax.fori_loop(0, NWP, wpair, 0)
        pltpu.make_async_copy(st3a, out_sl(YPT - 2), sem_wa).wait()
        pltpu.make_async_copy(st3b, out_sl(YPT - 1), sem_wb).wait()
        return carry

    lax.fori_loop(0, TPC, task_body, 0)


def kernel(updates, mask):
    u4 = updates.transpose(0, 1, 3, 2)
    m4 = mask.astype(jnp.int32).transpose(0, 1, 3, 2)
    out = _unpool_sc(u4, m4)
    return out.transpose(0, 1, 3, 2)
